# baseline (speedup 1.0000x reference)
import functools

import jax
import jax.numpy as jnp
import numpy as np
from jax import lax
from jax.experimental import pallas as pl
from jax.experimental.pallas import tpu as pltpu
from jax.experimental.pallas import tpu_sc as plsc

N_ATOMS = 1024
CAP = 32
LANES = 16
N_CHUNKS = N_ATOMS // LANES
NC = 2
NS = 16
NW = NC * NS
ROWS_PER_W = N_ATOMS // NW
BIG_INIT = np.float32(3.0e38)
CUT2 = np.float32(225.0)
BIG2 = np.float32(900.0)


def _full16(val, dtype=jnp.int32):
    return jnp.full((LANES,), val, dtype)


def _merge_topk(t0k, t0v, t1k, t1v, ck, cv):
    ck, cv = plsc.sort_key_val(ck, cv)
    rk = lax.rev(ck, (0,))
    rv = lax.rev(cv, (0,))
    sel = t1k <= rk
    lk = jnp.where(sel, t1k, rk)
    lv = jnp.where(sel, t1v, rv)
    lk, lv = plsc.sort_key_val(lk, lv)
    rlk = lax.rev(lk, (0,))
    rlv = lax.rev(lv, (0,))
    sel2 = t0k <= rlk
    ak = jnp.where(sel2, t0k, rlk)
    av = jnp.where(sel2, t0v, rlv)
    bk = jnp.where(sel2, rlk, t0k)
    bv = jnp.where(sel2, rlv, t0v)
    t0k, t0v = plsc.sort_key_val(ak, av)
    t1k, t1v = plsc.sort_key_val(bk, bv)
    return t0k, t0v, t1k, t1v


def _sc_kshell(xs, ls):
    mesh = plsc.VectorSubcoreMesh(core_axis_name="c", subcore_axis_name="s")

    @functools.partial(
        pl.kernel,
        out_type=[
            jax.ShapeDtypeStruct((N_ATOMS * CAP,), jnp.float32),
            jax.ShapeDtypeStruct((N_ATOMS * CAP,), jnp.int32),
            jax.ShapeDtypeStruct((NW * 3 * ROWS_PER_W * CAP,), jnp.float32),
        ],
        scratch_types=[
            pltpu.VMEM((3 * N_ATOMS,), jnp.float32),
            pltpu.VMEM((48,), jnp.float32),
            pltpu.VMEM((184,), jnp.float32),
            pltpu.VMEM((184,), jnp.int32),
            pltpu.VMEM((ROWS_PER_W * CAP,), jnp.float32),
            pltpu.VMEM((ROWS_PER_W * CAP,), jnp.int32),
            pltpu.VMEM((3 * ROWS_PER_W * CAP,), jnp.float32),
        ],
        mesh=mesh,
        compiler_params=pltpu.CompilerParams(needs_layout_passes=False),
    )
    def kfn(xs_hbm, ls_hbm, sq_hbm, j_hbm, v_hbm, xs_v, ls_v, kb_v, jb_v,
            ks_v, js_v, vs_v):
        wid = lax.axis_index("s") * NC + lax.axis_index("c")
        row0 = wid * ROWS_PER_W
        pltpu.sync_copy(xs_hbm, xs_v)
        pltpu.sync_copy(ls_hbm, ls_v)
        lv = [ls_v[pl.ds(16 * a, LANES)] for a in range(3)]
        iota3 = lax.iota(jnp.int32, LANES) * 3

        def do_row(r, _):
            gi = row0 + r
            gi_idx = _full16(gi)
            xi = [
                plsc.load_gather(xs_v, [gi_idx * 3 + a])
                for a in range(3)
            ]

            def consolidate(ops):
                t0k, t0v, t1k, t1v, cur = ops
                curv = jnp.full((LANES,), cur, jnp.int32)
                trip = (cur + 15) // 16

                def cbody(cc, tc):
                    c0k, c0v, c1k, c1v = tc
                    off = cc * 16
                    bk = kb_v[pl.ds(off, LANES)]
                    bv = jb_v[pl.ds(off, LANES)]
                    lane = lax.iota(jnp.int32, LANES) + off
                    bk = jnp.where(lane < curv, bk, BIG_INIT)
                    return _merge_topk(c0k, c0v, c1k, c1v, bk, bv)

                t0k, t0v, t1k, t1v = lax.fori_loop(
                    0, trip, cbody, (t0k, t0v, t1k, t1v))
                return t0k, t0v, t1k, t1v, jnp.max(t1k), 0

            def do_group(g, carry):
                t0k, t0v, t1k, t1v, thr, cur = carry
                gbase = g * (8 * LANES)
                for u in range(8):
                    base = gbase + u * LANES
                    sq = None
                    bidx = iota3 + (3 * base)
                    for a in range(3):
                        x0 = plsc.load_gather(xs_v, [bidx + a])
                        t0 = x0 - xi[a]
                        lsel = jnp.where(t0 > 0.0, -lv[a], lv[a])
                        ts = (x0 + lsel) - xi[a]
                        q = jnp.minimum(t0 * t0, ts * ts)
                        sq = q if sq is None else sq + q
                    jv = lax.iota(jnp.int32, LANES) + base
                    key = jnp.where(sq > CUT2, BIG2, sq)
                    key = jnp.where(jv == gi, BIG2, key)
                    pm = key < thr
                    plsc.store_compressed(
                        kb_v.at[pl.ds(cur, LANES)], key, mask=pm)
                    plsc.store_compressed(
                        jb_v.at[pl.ds(cur, LANES)], jv, mask=pm)
                    cnt = plsc.all_reduce_population_count(pm)
                    cur = cur + cnt[0]

                def kept(ops):
                    a0k, a0v, a1k, a1v, c = ops
                    return a0k, a0v, a1k, a1v, thr, c

                return lax.cond(
                    cur >= 48, consolidate, kept,
                    (t0k, t0v, t1k, t1v, cur),
                )

            init = (
                jnp.full((LANES,), BIG_INIT, jnp.float32),
                jnp.zeros((LANES,), jnp.int32),
                jnp.full((LANES,), BIG_INIT, jnp.float32),
                jnp.zeros((LANES,), jnp.int32),
                BIG_INIT,
                0,
            )
            carry = lax.fori_loop(0, N_CHUNKS // 8, do_group, init)
            t0k, t0v, t1k, t1v, _, _ = consolidate(
                (carry[0], carry[1], carry[2], carry[3], carry[5]))

            ks_v[pl.ds(r * CAP, LANES)] = t0k
            ks_v[pl.ds(r * CAP + LANES, LANES)] = t1k
            js_v[pl.ds(r * CAP, LANES)] = t0v
            js_v[pl.ds(r * CAP + LANES, LANES)] = t1v
            for h, jsel in ((0, t0v), (1, t1v)):
                for a in range(3):
                    g0 = plsc.load_gather(xs_v, [jsel * 3 + a])
                    t0 = g0 - xi[a]
                    tp = (g0 + lv[a]) - xi[a]
                    tm = (g0 - lv[a]) - xi[a]
                    q0 = t0 * t0
                    qp = tp * tp
                    qm = tm * tm
                    b = jnp.where(qp < q0, tp, t0)
                    qb = jnp.where(qp < q0, qp, q0)
                    b = jnp.where(qm < qb, tm, b)
                    vs_v[pl.ds(a * (ROWS_PER_W * CAP) + r * CAP + h * LANES,
                               LANES)] = b
            return 0

        lax.fori_loop(0, ROWS_PER_W, do_row, 0)
        pltpu.sync_copy(ks_v, sq_hbm.at[pl.ds(row0 * CAP, ROWS_PER_W * CAP)])
        pltpu.sync_copy(js_v, j_hbm.at[pl.ds(row0 * CAP, ROWS_PER_W * CAP)])
        for a in range(3):
            pltpu.sync_copy(
                vs_v.at[pl.ds(a * (ROWS_PER_W * CAP), ROWS_PER_W * CAP)],
                v_hbm.at[pl.ds(a * (N_ATOMS * CAP) + row0 * CAP,
                               ROWS_PER_W * CAP)])

    return kfn(xs, ls)


def _tc_post(sq, jidx, v0, v1, v2):
    def body(sq_r, j_r, v0_r, v1_r, v2_r, j_o, e_o, r0_o, r1_o, r2_o):
        d = jnp.sqrt(sq_r[...] + 1e-12)
        j = j_r[...]
        vs = [v0_r[...], v1_r[...], v2_r[...]]
        col = lax.broadcasted_iota(jnp.int32, d.shape, 1)

        def shl(x):
            return jnp.concatenate([x[:, 1:], x[:, :1]], axis=1)

        def shr(x):
            return jnp.concatenate([x[:, :1], x[:, :-1]], axis=1)

        for p in range(4):
            dn = shl(d)
            jl = shl(j)
            dp = shr(d)
            jr = shr(j)
            eq = (d == dn) & (j > jl) & (col % 2 == (p % 2)) & (col < CAP - 1)
            eqr = (dp == d) & (jr > j) & ((col + 1) % 2 == (p % 2)) & (col >= 1)
            j = jnp.where(eq, jl, jnp.where(eqr, jr, j))
            vs = [
                jnp.where(eq, shl(v), jnp.where(eqr, shr(v), v)) for v in vs
            ]
        dk = jnp.broadcast_to(d[:, 11:12], d.shape)
        mask = (d <= dk * (1.0 + 1e-6)) & (d < 15.0)
        j_o[...] = j
        e_o[...] = jnp.where(mask, d, 0.0)
        r0_o[...] = jnp.where(mask, vs[0], 0.0)
        r1_o[...] = jnp.where(mask, vs[1], 0.0)
        r2_o[...] = jnp.where(mask, vs[2], 0.0)

    return pl.pallas_call(
        body,
        out_shape=[
            jax.ShapeDtypeStruct((N_ATOMS, CAP), jnp.int32),
            jax.ShapeDtypeStruct((N_ATOMS, CAP), jnp.float32),
            jax.ShapeDtypeStruct((N_ATOMS, CAP), jnp.float32),
            jax.ShapeDtypeStruct((N_ATOMS, CAP), jnp.float32),
            jax.ShapeDtypeStruct((N_ATOMS, CAP), jnp.float32),
        ],
    )(sq, jidx, v0, v1, v2)


def kernel(frac, lattice, numbers):
    N = frac.shape[0]
    cart = frac @ lattice
    xs = cart.reshape(3 * N)
    ls = jnp.repeat(jnp.diagonal(lattice), LANES)

    sq_flat, j_flat, v_flat = _sc_kshell(xs, ls)

    sq = sq_flat.reshape(N, CAP)
    jidx = j_flat.reshape(N, CAP)
    v = v_flat.reshape(3, N, CAP)
    v0 = v[0]
    v1 = v[1]
    v2 = v[2]

    dst, edge_d, r0, r1, r2 = _tc_post(sq, jidx, v0, v1, v2)

    mask = edge_d > 0.0
    src = jnp.broadcast_to(jnp.arange(N, dtype=dst.dtype)[:, None], (N, CAP))
    r_vec = jnp.stack([r0, r1, r2], axis=-1)
    return src, dst, mask, r_vec, edge_d


# speedup vs baseline: 1.0200x; 1.0200x over previous
import functools

import jax
import jax.numpy as jnp
import numpy as np
from jax import lax
from jax.experimental import pallas as pl
from jax.experimental.pallas import tpu as pltpu
from jax.experimental.pallas import tpu_sc as plsc

N_ATOMS = 1024
CAP = 32
LANES = 16
N_CHUNKS = N_ATOMS // LANES
NC = 2
NS = 16
NW = NC * NS
ROWS_PER_W = N_ATOMS // NW
BIG_INIT = np.float32(3.0e38)
CUT2 = np.float32(225.0)
BIG2 = np.float32(900.0)


def _full16(val, dtype=jnp.int32):
    return jnp.full((LANES,), val, dtype)


def _merge_topk(t0k, t0v, t1k, t1v, ck, cv):
    ck, cv = plsc.sort_key_val(ck, cv)
    rk = lax.rev(ck, (0,))
    rv = lax.rev(cv, (0,))
    sel = t1k <= rk
    lk = jnp.where(sel, t1k, rk)
    lv = jnp.where(sel, t1v, rv)
    lk, lv = plsc.sort_key_val(lk, lv)
    rlk = lax.rev(lk, (0,))
    rlv = lax.rev(lv, (0,))
    sel2 = t0k <= rlk
    ak = jnp.where(sel2, t0k, rlk)
    av = jnp.where(sel2, t0v, rlv)
    bk = jnp.where(sel2, rlk, t0k)
    bv = jnp.where(sel2, rlv, t0v)
    t0k, t0v = plsc.sort_key_val(ak, av)
    t1k, t1v = plsc.sort_key_val(bk, bv)
    return t0k, t0v, t1k, t1v


def _sc_kshell(xs, ls):
    mesh = plsc.VectorSubcoreMesh(core_axis_name="c", subcore_axis_name="s")

    @functools.partial(
        pl.kernel,
        out_type=[
            jax.ShapeDtypeStruct((N_ATOMS * CAP,), jnp.float32),
            jax.ShapeDtypeStruct((N_ATOMS * CAP,), jnp.int32),
            jax.ShapeDtypeStruct((NW * 3 * ROWS_PER_W * CAP,), jnp.float32),
        ],
        scratch_types=[
            pltpu.VMEM((3 * N_ATOMS,), jnp.float32),
            pltpu.VMEM((48,), jnp.float32),
            pltpu.VMEM((184,), jnp.float32),
            pltpu.VMEM((184,), jnp.int32),
            pltpu.VMEM((ROWS_PER_W * CAP,), jnp.float32),
            pltpu.VMEM((ROWS_PER_W * CAP,), jnp.int32),
            pltpu.VMEM((3 * ROWS_PER_W * CAP,), jnp.float32),
        ],
        mesh=mesh,
        compiler_params=pltpu.CompilerParams(needs_layout_passes=False),
    )
    def kfn(xs_hbm, ls_hbm, sq_hbm, j_hbm, v_hbm, xs_v, ls_v, kb_v, jb_v,
            ks_v, js_v, vs_v):
        wid = lax.axis_index("s") * NC + lax.axis_index("c")
        row0 = wid * ROWS_PER_W
        pltpu.sync_copy(xs_hbm, xs_v)
        pltpu.sync_copy(ls_hbm, ls_v)
        lv = [ls_v[pl.ds(16 * a, LANES)] for a in range(3)]

        def do_row(r, _):
            gi = row0 + r
            gi_idx = _full16(gi)
            xi = [
                plsc.load_gather(xs_v, [gi_idx + a * N_ATOMS])
                for a in range(3)
            ]

            def consolidate(ops):
                t0k, t0v, t1k, t1v, cur = ops
                curv = jnp.full((LANES,), cur, jnp.int32)
                trip = (cur + 15) // 16

                def cbody(cc, tc):
                    c0k, c0v, c1k, c1v = tc
                    off = cc * 16
                    bk = kb_v[pl.ds(off, LANES)]
                    bv = jb_v[pl.ds(off, LANES)]
                    lane = lax.iota(jnp.int32, LANES) + off
                    bk = jnp.where(lane < curv, bk, BIG_INIT)
                    return _merge_topk(c0k, c0v, c1k, c1v, bk, bv)

                t0k, t0v, t1k, t1v = lax.fori_loop(
                    0, trip, cbody, (t0k, t0v, t1k, t1v))
                return t0k, t0v, t1k, t1v, jnp.max(t1k), 0

            def do_group(g, carry):
                t0k, t0v, t1k, t1v, thr, cur = carry
                gbase = g * (8 * LANES)
                for u in range(8):
                    base = gbase + u * LANES
                    sq = None
                    for a in range(3):
                        x0 = xs_v[pl.ds(a * N_ATOMS + base, LANES)]
                        t0 = x0 - xi[a]
                        lsel = jnp.where(t0 > 0.0, -lv[a], lv[a])
                        ts = (x0 + lsel) - xi[a]
                        q = jnp.minimum(t0 * t0, ts * ts)
                        sq = q if sq is None else sq + q
                    jv = lax.iota(jnp.int32, LANES) + base
                    key = jnp.where(sq > CUT2, BIG2, sq)
                    key = jnp.where(jv == gi, BIG2, key)
                    pm = key < thr
                    plsc.store_compressed(
                        kb_v.at[pl.ds(cur, LANES)], key, mask=pm)
                    plsc.store_compressed(
                        jb_v.at[pl.ds(cur, LANES)], jv, mask=pm)
                    cnt = plsc.all_reduce_population_count(pm)
                    cur = cur + cnt[0]

                def kept(ops):
                    a0k, a0v, a1k, a1v, c = ops
                    return a0k, a0v, a1k, a1v, thr, c

                return lax.cond(
                    cur >= 48, consolidate, kept,
                    (t0k, t0v, t1k, t1v, cur),
                )

            init = (
                jnp.full((LANES,), BIG_INIT, jnp.float32),
                jnp.zeros((LANES,), jnp.int32),
                jnp.full((LANES,), BIG_INIT, jnp.float32),
                jnp.zeros((LANES,), jnp.int32),
                BIG_INIT,
                0,
            )
            carry = lax.fori_loop(0, N_CHUNKS // 8, do_group, init)
            t0k, t0v, t1k, t1v, _, _ = consolidate(
                (carry[0], carry[1], carry[2], carry[3], carry[5]))

            ks_v[pl.ds(r * CAP, LANES)] = t0k
            ks_v[pl.ds(r * CAP + LANES, LANES)] = t1k
            js_v[pl.ds(r * CAP, LANES)] = t0v
            js_v[pl.ds(r * CAP + LANES, LANES)] = t1v
            for h, jsel in ((0, t0v), (1, t1v)):
                for a in range(3):
                    g0 = plsc.load_gather(xs_v, [jsel + a * N_ATOMS])
                    t0 = g0 - xi[a]
                    tp = (g0 + lv[a]) - xi[a]
                    tm = (g0 - lv[a]) - xi[a]
                    q0 = t0 * t0
                    qp = tp * tp
                    qm = tm * tm
                    b = jnp.where(qp < q0, tp, t0)
                    qb = jnp.where(qp < q0, qp, q0)
                    b = jnp.where(qm < qb, tm, b)
                    vs_v[pl.ds(a * (ROWS_PER_W * CAP) + r * CAP + h * LANES,
                               LANES)] = b
            return 0

        lax.fori_loop(0, ROWS_PER_W, do_row, 0)
        pltpu.sync_copy(ks_v, sq_hbm.at[pl.ds(row0 * CAP, ROWS_PER_W * CAP)])
        pltpu.sync_copy(js_v, j_hbm.at[pl.ds(row0 * CAP, ROWS_PER_W * CAP)])
        for a in range(3):
            pltpu.sync_copy(
                vs_v.at[pl.ds(a * (ROWS_PER_W * CAP), ROWS_PER_W * CAP)],
                v_hbm.at[pl.ds(a * (N_ATOMS * CAP) + row0 * CAP,
                               ROWS_PER_W * CAP)])

    return kfn(xs, ls)


def _tc_post(sq, jidx, v0, v1, v2):
    def body(sq_r, j_r, v0_r, v1_r, v2_r, j_o, e_o, r0_o, r1_o, r2_o):
        d = jnp.sqrt(sq_r[...] + 1e-12)
        j = j_r[...]
        vs = [v0_r[...], v1_r[...], v2_r[...]]
        col = lax.broadcasted_iota(jnp.int32, d.shape, 1)

        def shl(x):
            return jnp.concatenate([x[:, 1:], x[:, :1]], axis=1)

        def shr(x):
            return jnp.concatenate([x[:, :1], x[:, :-1]], axis=1)

        for p in range(4):
            dn = shl(d)
            jl = shl(j)
            dp = shr(d)
            jr = shr(j)
            eq = (d == dn) & (j > jl) & (col % 2 == (p % 2)) & (col < CAP - 1)
            eqr = (dp == d) & (jr > j) & ((col + 1) % 2 == (p % 2)) & (col >= 1)
            j = jnp.where(eq, jl, jnp.where(eqr, jr, j))
            vs = [
                jnp.where(eq, shl(v), jnp.where(eqr, shr(v), v)) for v in vs
            ]
        dk = jnp.broadcast_to(d[:, 11:12], d.shape)
        mask = (d <= dk * (1.0 + 1e-6)) & (d < 15.0)
        j_o[...] = j
        e_o[...] = jnp.where(mask, d, 0.0)
        r0_o[...] = jnp.where(mask, vs[0], 0.0)
        r1_o[...] = jnp.where(mask, vs[1], 0.0)
        r2_o[...] = jnp.where(mask, vs[2], 0.0)

    return pl.pallas_call(
        body,
        out_shape=[
            jax.ShapeDtypeStruct((N_ATOMS, CAP), jnp.int32),
            jax.ShapeDtypeStruct((N_ATOMS, CAP), jnp.float32),
            jax.ShapeDtypeStruct((N_ATOMS, CAP), jnp.float32),
            jax.ShapeDtypeStruct((N_ATOMS, CAP), jnp.float32),
            jax.ShapeDtypeStruct((N_ATOMS, CAP), jnp.float32),
        ],
    )(sq, jidx, v0, v1, v2)


def kernel(frac, lattice, numbers):
    N = frac.shape[0]
    cart = frac @ lattice
    xs = cart.T.reshape(3 * N)
    ls = jnp.repeat(jnp.diagonal(lattice), LANES)

    sq_flat, j_flat, v_flat = _sc_kshell(xs, ls)

    sq = sq_flat.reshape(N, CAP)
    jidx = j_flat.reshape(N, CAP)
    v = v_flat.reshape(3, N, CAP)
    v0 = v[0]
    v1 = v[1]
    v2 = v[2]

    dst, edge_d, r0, r1, r2 = _tc_post(sq, jidx, v0, v1, v2)

    mask = edge_d > 0.0
    src = jnp.broadcast_to(jnp.arange(N, dtype=dst.dtype)[:, None], (N, CAP))
    r_vec = jnp.stack([r0, r1, r2], axis=-1)
    return src, dst, mask, r_vec, edge_d


# speedup vs baseline: 1.0381x; 1.0177x over previous
import functools

import jax
import jax.numpy as jnp
import numpy as np
from jax import lax
from jax.experimental import pallas as pl
from jax.experimental.pallas import tpu as pltpu
from jax.experimental.pallas import tpu_sc as plsc

N_ATOMS = 1024
CAP = 32
LANES = 16
N_CHUNKS = N_ATOMS // LANES
NC = 2
NS = 16
NW = NC * NS
ROWS_PER_W = N_ATOMS // NW
BIG_INIT = np.float32(3.0e38)
CUT2 = np.float32(225.0)
BIG2 = np.float32(900.0)


def _full16(val, dtype=jnp.int32):
    return jnp.full((LANES,), val, dtype)


def _merge_topk(t0k, t0v, t1k, t1v, ck, cv):
    ck, cv = plsc.sort_key_val(ck, cv)
    rk = lax.rev(ck, (0,))
    rv = lax.rev(cv, (0,))
    sel = t1k <= rk
    lk = jnp.where(sel, t1k, rk)
    lv = jnp.where(sel, t1v, rv)
    lk, lv = plsc.sort_key_val(lk, lv)
    rlk = lax.rev(lk, (0,))
    rlv = lax.rev(lv, (0,))
    sel2 = t0k <= rlk
    ak = jnp.where(sel2, t0k, rlk)
    av = jnp.where(sel2, t0v, rlv)
    bk = jnp.where(sel2, rlk, t0k)
    bv = jnp.where(sel2, rlv, t0v)
    t0k, t0v = plsc.sort_key_val(ak, av)
    t1k, t1v = plsc.sort_key_val(bk, bv)
    return t0k, t0v, t1k, t1v


def _sc_kshell(xs, ls):
    mesh = plsc.VectorSubcoreMesh(core_axis_name="c", subcore_axis_name="s")

    @functools.partial(
        pl.kernel,
        out_type=[
            jax.ShapeDtypeStruct((N_ATOMS * CAP,), jnp.float32),
            jax.ShapeDtypeStruct((N_ATOMS * CAP,), jnp.int32),
            jax.ShapeDtypeStruct((NW * 3 * ROWS_PER_W * CAP,), jnp.float32),
        ],
        scratch_types=[
            pltpu.VMEM((3 * N_ATOMS,), jnp.float32),
            pltpu.VMEM((48,), jnp.float32),
            pltpu.VMEM((184,), jnp.float32),
            pltpu.VMEM((184,), jnp.int32),
            pltpu.VMEM((ROWS_PER_W * CAP,), jnp.float32),
            pltpu.VMEM((ROWS_PER_W * CAP,), jnp.int32),
            pltpu.VMEM((3 * ROWS_PER_W * CAP,), jnp.float32),
        ],
        mesh=mesh,
        compiler_params=pltpu.CompilerParams(needs_layout_passes=False),
    )
    def kfn(xs_hbm, ls_hbm, sq_hbm, j_hbm, v_hbm, xs_v, ls_v, kb_v, jb_v,
            ks_v, js_v, vs_v):
        wid = lax.axis_index("s") * NC + lax.axis_index("c")
        row0 = wid * ROWS_PER_W
        pltpu.sync_copy(xs_hbm, xs_v)
        pltpu.sync_copy(ls_hbm, ls_v)
        lv = [ls_v[pl.ds(16 * a, LANES)] for a in range(3)]

        def do_row(r, _):
            gi = row0 + r
            gi_idx = _full16(gi)
            xi = [
                plsc.load_gather(xs_v, [gi_idx + a * N_ATOMS])
                for a in range(3)
            ]

            def consolidate(ops):
                t0k, t0v, t1k, t1v, cur = ops
                curv = jnp.full((LANES,), cur, jnp.int32)
                trip = (cur + 15) // 16

                def cbody(cc, tc):
                    c0k, c0v, c1k, c1v = tc
                    off = cc * 16
                    bk = kb_v[pl.ds(off, LANES)]
                    bv = jb_v[pl.ds(off, LANES)]
                    lane = lax.iota(jnp.int32, LANES) + off
                    bk = jnp.where(lane < curv, bk, BIG_INIT)
                    return _merge_topk(c0k, c0v, c1k, c1v, bk, bv)

                t0k, t0v, t1k, t1v = lax.fori_loop(
                    0, trip, cbody, (t0k, t0v, t1k, t1v))
                return t0k, t0v, t1k, t1v, jnp.max(t1k), 0

            def do_group(g, carry):
                t0k, t0v, t1k, t1v, thr, cur = carry
                gbase = g * (8 * LANES)
                for u in range(8):
                    base = gbase + u * LANES
                    sq = None
                    for a in range(3):
                        x0 = xs_v[pl.ds(a * N_ATOMS + base, LANES)]
                        t0 = x0 - xi[a]
                        lsel = jnp.where(t0 > 0.0, -lv[a], lv[a])
                        ts = (x0 + lsel) - xi[a]
                        q = jnp.minimum(t0 * t0, ts * ts)
                        sq = q if sq is None else sq + q
                    jv = lax.iota(jnp.int32, LANES) + base
                    key = jnp.where(sq > CUT2, BIG2, sq)
                    key = jnp.where(jv == gi, BIG2, key)
                    pm = key < thr
                    plsc.store_compressed(
                        kb_v.at[pl.ds(cur, LANES)], key, mask=pm)
                    plsc.store_compressed(
                        jb_v.at[pl.ds(cur, LANES)], jv, mask=pm)
                    cnt = plsc.all_reduce_population_count(pm)
                    cur = cur + cnt[0]

                def kept(ops):
                    a0k, a0v, a1k, a1v, c = ops
                    return a0k, a0v, a1k, a1v, thr, c

                return lax.cond(
                    cur >= 48, consolidate, kept,
                    (t0k, t0v, t1k, t1v, cur),
                )

            init = (
                jnp.full((LANES,), BIG_INIT, jnp.float32),
                jnp.zeros((LANES,), jnp.int32),
                jnp.full((LANES,), BIG_INIT, jnp.float32),
                jnp.zeros((LANES,), jnp.int32),
                BIG_INIT,
                0,
            )
            carry = lax.fori_loop(0, N_CHUNKS // 8, do_group, init)
            t0k, t0v, t1k, t1v, _, _ = consolidate(
                (carry[0], carry[1], carry[2], carry[3], carry[5]))

            ks_v[pl.ds(r * CAP, LANES)] = t0k
            ks_v[pl.ds(r * CAP + LANES, LANES)] = t1k
            js_v[pl.ds(r * CAP, LANES)] = t0v
            js_v[pl.ds(r * CAP + LANES, LANES)] = t1v
            for h, jsel in ((0, t0v), (1, t1v)):
                for a in range(3):
                    g0 = plsc.load_gather(xs_v, [jsel + a * N_ATOMS])
                    t0 = g0 - xi[a]
                    tp = (g0 + lv[a]) - xi[a]
                    tm = (g0 - lv[a]) - xi[a]
                    q0 = t0 * t0
                    qp = tp * tp
                    qm = tm * tm
                    b = jnp.where(qp < q0, tp, t0)
                    qb = jnp.where(qp < q0, qp, q0)
                    b = jnp.where(qm < qb, tm, b)
                    vs_v[pl.ds(a * (ROWS_PER_W * CAP) + r * CAP + h * LANES,
                               LANES)] = b
            return 0

        lax.fori_loop(0, ROWS_PER_W, do_row, 0)
        pltpu.sync_copy(ks_v, sq_hbm.at[pl.ds(row0 * CAP, ROWS_PER_W * CAP)])
        pltpu.sync_copy(js_v, j_hbm.at[pl.ds(row0 * CAP, ROWS_PER_W * CAP)])
        pltpu.sync_copy(
            vs_v, v_hbm.at[pl.ds(wid * 3 * ROWS_PER_W * CAP,
                                 3 * ROWS_PER_W * CAP)])

    return kfn(xs, ls)


def _tc_post(sq, jidx, v0, v1, v2):
    def body(sq_r, j_r, v0_r, v1_r, v2_r, j_o, e_o, r0_o, r1_o, r2_o):
        d = jnp.sqrt(sq_r[...] + 1e-12)
        j = j_r[...]
        vs = [v0_r[...], v1_r[...], v2_r[...]]
        col = lax.broadcasted_iota(jnp.int32, d.shape, 1)

        def shl(x):
            return jnp.concatenate([x[:, 1:], x[:, :1]], axis=1)

        def shr(x):
            return jnp.concatenate([x[:, :1], x[:, :-1]], axis=1)

        for p in range(4):
            dn = shl(d)
            jl = shl(j)
            dp = shr(d)
            jr = shr(j)
            eq = (d == dn) & (j > jl) & (col % 2 == (p % 2)) & (col < CAP - 1)
            eqr = (dp == d) & (jr > j) & ((col + 1) % 2 == (p % 2)) & (col >= 1)
            j = jnp.where(eq, jl, jnp.where(eqr, jr, j))
            vs = [
                jnp.where(eq, shl(v), jnp.where(eqr, shr(v), v)) for v in vs
            ]
        dk = jnp.broadcast_to(d[:, 11:12], d.shape)
        mask = (d <= dk * (1.0 + 1e-6)) & (d < 15.0)
        j_o[...] = j
        e_o[...] = jnp.where(mask, d, 0.0)
        r0_o[...] = jnp.where(mask, vs[0], 0.0)
        r1_o[...] = jnp.where(mask, vs[1], 0.0)
        r2_o[...] = jnp.where(mask, vs[2], 0.0)

    return pl.pallas_call(
        body,
        out_shape=[
            jax.ShapeDtypeStruct((N_ATOMS, CAP), jnp.int32),
            jax.ShapeDtypeStruct((N_ATOMS, CAP), jnp.float32),
            jax.ShapeDtypeStruct((N_ATOMS, CAP), jnp.float32),
            jax.ShapeDtypeStruct((N_ATOMS, CAP), jnp.float32),
            jax.ShapeDtypeStruct((N_ATOMS, CAP), jnp.float32),
        ],
    )(sq, jidx, v0, v1, v2)


def kernel(frac, lattice, numbers):
    N = frac.shape[0]
    cart = frac @ lattice
    xs = cart.T.reshape(3 * N)
    ls = jnp.repeat(jnp.diagonal(lattice), LANES)

    sq_flat, j_flat, v_flat = _sc_kshell(xs, ls)

    sq = sq_flat.reshape(N, CAP)
    jidx = j_flat.reshape(N, CAP)
    v = v_flat.reshape(NW, 3, ROWS_PER_W, CAP)
    v0 = v[:, 0].reshape(N, CAP)
    v1 = v[:, 1].reshape(N, CAP)
    v2 = v[:, 2].reshape(N, CAP)

    dst, edge_d, r0, r1, r2 = _tc_post(sq, jidx, v0, v1, v2)

    mask = edge_d > 0.0
    src = jnp.broadcast_to(jnp.arange(N, dtype=dst.dtype)[:, None], (N, CAP))
    r_vec = jnp.stack([r0, r1, r2], axis=-1)
    return src, dst, mask, r_vec, edge_d
